# Initial kernel scaffold; baseline (speedup 1.0000x reference)
import jax
import jax.numpy as jnp
from jax.experimental import pallas as pl


def kernel(inputs, pe_table):
    raise NotImplementedError("write your pallas kernel here")


# speedup vs baseline: 2.1865x; 2.1865x over previous
import functools

import jax
import jax.numpy as jnp
from jax import lax
from jax.experimental import pallas as pl
from jax.experimental.pallas import tpu as pltpu
from jax.experimental.pallas import tpu_sc as plsc

D_MODEL = 1024
NC = 2
NS = 16
NW = NC * NS


@functools.lru_cache(maxsize=None)
def _make_gather(B, C):
    BPW = B // NW
    NCHUNK = BPW // C
    mesh = plsc.VectorSubcoreMesh(core_axis_name="c", subcore_axis_name="s")

    @functools.partial(
        pl.kernel,
        mesh=mesh,
        out_type=jax.ShapeDtypeStruct((B, D_MODEL), jnp.float32),
        scratch_types=[
            pltpu.VMEM((BPW,), jnp.int32),
            pltpu.VMEM((C, D_MODEL), jnp.float32),
            pltpu.SemaphoreType.DMA,
        ],
    )
    def body(idx_hbm, table_hbm, out_hbm, idx_v, rows_v, sem):
        wid = lax.axis_index("s") * NC + lax.axis_index("c")
        base = wid * BPW
        pltpu.sync_copy(idx_hbm.at[pl.ds(base, BPW)], idx_v)

        def chunk(i, carry):
            pltpu.async_copy(
                table_hbm.at[idx_v.at[pl.ds(i * C, C)]], rows_v, sem
            ).wait()
            pltpu.sync_copy(rows_v, out_hbm.at[pl.ds(base + i * C, C)])
            return carry

        lax.fori_loop(0, NCHUNK, chunk, 0)

    return body


def kernel(inputs, pe_table):
    B = inputs.size
    flat = inputs.reshape(B)
    out = _make_gather(B, 64)(flat, pe_table)
    return out.reshape(inputs.shape + (D_MODEL,))


# speedup vs baseline: 2.3917x; 1.0939x over previous
import functools

import jax
import jax.numpy as jnp
from jax import lax
from jax.experimental import pallas as pl
from jax.experimental.pallas import tpu as pltpu
from jax.experimental.pallas import tpu_sc as plsc

D_MODEL = 1024
NC = 2
NS = 16
NW = NC * NS


@functools.lru_cache(maxsize=None)
def _make_gather(B, C, NBUF):
    BPW = B // NW
    NCHUNK = BPW // C
    NG = NCHUNK // NBUF
    mesh = plsc.VectorSubcoreMesh(core_axis_name="c", subcore_axis_name="s")

    @functools.partial(
        pl.kernel,
        mesh=mesh,
        out_type=jax.ShapeDtypeStruct((B, D_MODEL), jnp.float32),
        scratch_types=[
            pltpu.VMEM((BPW,), jnp.int32),
            pltpu.VMEM((NBUF, C, D_MODEL), jnp.float32),
        ]
        + [pltpu.SemaphoreType.DMA] * (2 * NBUF),
    )
    def body(idx_hbm, table_hbm, out_hbm, idx_v, rows_v, *sems):
        gsem, osem = sems[:NBUF], sems[NBUF:]
        wid = lax.axis_index("s") * NC + lax.axis_index("c")
        base = wid * BPW
        pltpu.sync_copy(idx_hbm.at[pl.ds(base, BPW)], idx_v)

        def start_gather(i, b):
            pltpu.async_copy(
                table_hbm.at[idx_v.at[pl.ds(i * C, C)]], rows_v.at[b], gsem[b]
            )

        def wait_gather(i, b):
            pltpu.make_async_copy(
                table_hbm.at[idx_v.at[pl.ds(i * C, C)]], rows_v.at[b], gsem[b]
            ).wait()

        def start_out(i, b):
            pltpu.async_copy(
                rows_v.at[b], out_hbm.at[pl.ds(base + i * C, C)], osem[b]
            )

        def wait_out(i, b):
            pltpu.make_async_copy(
                rows_v.at[b], out_hbm.at[pl.ds(base + i * C, C)], osem[b]
            ).wait()

        for b in range(NBUF):
            start_gather(b, b)

        def group(g, carry):
            for b in range(NBUF):
                i = g * NBUF + b
                wait_gather(i, b)
                start_out(i, b)
                wait_out(i, b)
                start_gather(i + NBUF, b)
            return carry

        lax.fori_loop(0, NG - 1, group, 0)

        for b in range(NBUF):
            i = (NG - 1) * NBUF + b
            wait_gather(i, b)
            start_out(i, b)
        for b in range(NBUF):
            i = (NG - 1) * NBUF + b
            wait_out(i, b)

    return body


def kernel(inputs, pe_table):
    B = inputs.size
    flat = inputs.reshape(B)
    out = _make_gather(B, 16, 4)(flat, pe_table)
    return out.reshape(inputs.shape + (D_MODEL,))
